# Initial kernel scaffold; baseline (speedup 1.0000x reference)
import jax
import jax.numpy as jnp
from jax.experimental import pallas as pl


def kernel(x, edge_index, edge_attr, batch, W_in, b_in, We, be, W1, b1, W2, b2, gamma, beta, Wr1, br1, Wr2, br2):
    raise NotImplementedError("write your pallas kernel here")


# speedup vs baseline: 3.1580x; 3.1580x over previous
import functools

import jax
import jax.numpy as jnp
from jax import lax
from jax.experimental import pallas as pl
from jax.experimental.pallas import tpu as pltpu
from jax.experimental.pallas import tpu_sc as plsc

_NUM_SC = 2
_NUM_SUBCORES = 16
_LANES = 16
_G = 64


def _mm_bias(x, W, b, block_rows):
    n, k = x.shape
    m = W.shape[1]
    assert n % block_rows == 0

    def body(x_ref, w_ref, b_ref, o_ref):
        o_ref[...] = (
            jnp.dot(x_ref[...], w_ref[...], preferred_element_type=jnp.float32)
            + b_ref[...]
        )

    return pl.pallas_call(
        body,
        grid=(n // block_rows,),
        in_specs=[
            pl.BlockSpec((block_rows, k), lambda i: (i, 0)),
            pl.BlockSpec((k, m), lambda i: (0, 0)),
            pl.BlockSpec((1, m), lambda i: (0, 0)),
        ],
        out_specs=pl.BlockSpec((block_rows, m), lambda i: (i, 0)),
        out_shape=jax.ShapeDtypeStruct((n, m), jnp.float32),
    )(x, W, b.reshape(1, m))


def _node_mlp_stats(h, parts, W1, b1, W2, b2, block_rows):
    n, d = h.shape

    def body(h_ref, a0_ref, a1_ref, w1_ref, b1_ref, w2_ref, b2_ref,
             z2_ref, st_ref):
        z = h_ref[...] + a0_ref[0] + a1_ref[0]
        t = jnp.maximum(
            jnp.dot(z, w1_ref[...], preferred_element_type=jnp.float32)
            + b1_ref[...], 0.0)
        z2 = (jnp.dot(t, w2_ref[...], preferred_element_type=jnp.float32)
              + b2_ref[...])
        z2_ref[...] = z2
        s = jnp.sum(z2, axis=0, keepdims=True)
        ss = jnp.sum(z2 * z2, axis=0, keepdims=True)
        st = jnp.concatenate([s, ss, jnp.zeros((6, d), jnp.float32)], axis=0)

        @pl.when(pl.program_id(0) == 0)
        def _():
            st_ref[...] = st

        @pl.when(pl.program_id(0) != 0)
        def _():
            st_ref[...] += st

    return pl.pallas_call(
        body,
        grid=(n // block_rows,),
        in_specs=[
            pl.BlockSpec((block_rows, d), lambda i: (i, 0)),
            pl.BlockSpec((1, block_rows, d), lambda i: (0, i, 0)),
            pl.BlockSpec((1, block_rows, d), lambda i: (1, i, 0)),
            pl.BlockSpec((d, d), lambda i: (0, 0)),
            pl.BlockSpec((1, d), lambda i: (0, 0)),
            pl.BlockSpec((d, d), lambda i: (0, 0)),
            pl.BlockSpec((1, d), lambda i: (0, 0)),
        ],
        out_specs=[
            pl.BlockSpec((block_rows, d), lambda i: (i, 0)),
            pl.BlockSpec((8, d), lambda i: (0, 0)),
        ],
        out_shape=[
            jax.ShapeDtypeStruct((n, d), jnp.float32),
            jax.ShapeDtypeStruct((8, d), jnp.float32),
        ],
    )(h, parts, parts, W1, b1.reshape(1, d), W2, b2.reshape(1, d))


def _bn_relu(z2, stats, gamma_l, beta_l, block_rows):
    n, d = z2.shape
    inv_n = 1.0 / n

    def body(z_ref, st_ref, g_ref, b_ref, o_ref):
        mu = st_ref[0, :] * inv_n
        var = st_ref[1, :] * inv_n - mu * mu
        scale = g_ref[...] * lax.rsqrt(var + 1e-5)[None, :]
        o_ref[...] = jnp.maximum(
            (z_ref[...] - mu[None, :]) * scale + b_ref[...], 0.0)

    return pl.pallas_call(
        body,
        grid=(n // block_rows,),
        in_specs=[
            pl.BlockSpec((block_rows, d), lambda i: (i, 0)),
            pl.BlockSpec((8, d), lambda i: (0, 0)),
            pl.BlockSpec((1, d), lambda i: (0, 0)),
            pl.BlockSpec((1, d), lambda i: (0, 0)),
        ],
        out_specs=pl.BlockSpec((block_rows, d), lambda i: (i, 0)),
        out_shape=jax.ShapeDtypeStruct((n, d), jnp.float32),
    )(z2, stats, gamma_l.reshape(1, d), beta_l.reshape(1, d))


def _pool(h, batch3, block_rows, num_graphs):
    n, d = h.shape
    nblk = n // block_rows

    def body(h_ref, b_ref, o_ref):
        bi = b_ref[0, 0, :]
        oh = (bi[:, None]
              == lax.broadcasted_iota(jnp.int32, (block_rows, num_graphs), 1)
              ).astype(jnp.float32)
        contrib = lax.dot_general(
            oh, h_ref[...], (((0,), (0,)), ((), ())),
            preferred_element_type=jnp.float32)

        @pl.when(pl.program_id(0) == 0)
        def _():
            o_ref[...] = contrib

        @pl.when(pl.program_id(0) != 0)
        def _():
            o_ref[...] += contrib

    return pl.pallas_call(
        body,
        grid=(nblk,),
        in_specs=[
            pl.BlockSpec((block_rows, d), lambda i: (i, 0)),
            pl.BlockSpec((1, 1, block_rows), lambda i: (i, 0, 0)),
        ],
        out_specs=pl.BlockSpec((num_graphs, d), lambda i: (0, 0)),
        out_shape=jax.ShapeDtypeStruct((num_graphs, d), jnp.float32),
    )(h, batch3)


def _readout(pooled, Wr1, br1, Wr2p, br2p):
    g, d = pooled.shape

    def body(p_ref, w1_ref, b1_ref, w2_ref, b2_ref, o_ref):
        t = jnp.maximum(
            jnp.dot(p_ref[...], w1_ref[...], preferred_element_type=jnp.float32)
            + b1_ref[...], 0.0)
        o_ref[...] = (
            jnp.dot(t, w2_ref[...], preferred_element_type=jnp.float32)
            + b2_ref[...])

    return pl.pallas_call(
        body,
        grid=(1,),
        in_specs=[
            pl.BlockSpec((g, d), lambda i: (0, 0)),
            pl.BlockSpec((d, d), lambda i: (0, 0)),
            pl.BlockSpec((1, d), lambda i: (0, 0)),
            pl.BlockSpec((d, d), lambda i: (0, 0)),
            pl.BlockSpec((1, d), lambda i: (0, 0)),
        ],
        out_specs=pl.BlockSpec((g, d), lambda i: (0, 0)),
        out_shape=jax.ShapeDtypeStruct((g, d), jnp.float32),
    )(pooled, Wr1, br1.reshape(1, d), Wr2p, br2p)


def _sc_aggregate(h, e, edge_index, zeros_nd):
    n, d = h.shape
    n_pad = zeros_nd.shape[0]
    num_edges = edge_index.shape[1]
    K = 128
    num_chunks = num_edges // K
    nw = _NUM_SC * _NUM_SUBCORES
    rows_per_tile = n_pad // _NUM_SUBCORES
    base_chunks = num_chunks // nw
    extra = num_chunks - base_chunks * nw
    q8 = d // _LANES

    mesh = plsc.VectorSubcoreMesh(
        core_axis_name="c", subcore_axis_name="s",
        num_cores=_NUM_SC, num_subcores=_NUM_SUBCORES)

    @functools.partial(
        pl.kernel,
        out_type=jax.ShapeDtypeStruct((_NUM_SC, n_pad, d), jnp.float32),
        mesh=mesh,
        scratch_types=[
            pltpu.VMEM_SHARED((n_pad, d), jnp.float32),
            pltpu.VMEM((2, K), jnp.int32),
            pltpu.VMEM((K, d), jnp.float32),
            pltpu.VMEM((K, d), jnp.float32),
            pltpu.SemaphoreType.DMA,
        ],
    )
    def agg(h_hbm, e_hbm, ei_hbm, z_hbm, out_hbm, acc, idx_v, hrows, ebuf, gsem):
        core = lax.axis_index("c")
        sub = lax.axis_index("s")
        wid = core * _NUM_SUBCORES + sub
        r0 = sub * rows_per_tile

        pltpu.sync_copy(z_hbm.at[pl.ds(r0, rows_per_tile)],
                        acc.at[pl.ds(r0, rows_per_tile)])
        plsc.subcore_barrier()

        my_chunks = base_chunks + jnp.where(wid < extra, 1, 0)

        def chunk_body(ci, carry):
            chunk = ci * nw + wid
            base = chunk * K
            pltpu.sync_copy(ei_hbm.at[:, pl.ds(base, K)], idx_v)
            pltpu.sync_copy(e_hbm.at[pl.ds(base, K)], ebuf)
            pltpu.async_copy(h_hbm.at[idx_v.at[0]], hrows, gsem).wait()

            def row_body(r, c2):
                for q in range(q8):
                    s = pl.ds(q * _LANES, _LANES)
                    ebuf[r, s] = jnp.maximum(hrows[r, s] + ebuf[r, s], 0.0)
                return c2

            lax.fori_loop(0, K, row_body, 0)
            pltpu.sync_copy(ebuf, acc.at[idx_v.at[1]], add=True)
            return carry

        lax.fori_loop(0, my_chunks, chunk_body, 0)
        plsc.subcore_barrier()

        pltpu.sync_copy(acc.at[pl.ds(r0, rows_per_tile)],
                        out_hbm.at[core, pl.ds(r0, rows_per_tile)])

    return agg(h, e, edge_index, zeros_nd)


def kernel(x, edge_index, edge_attr, batch, W_in, b_in, We, be, W1, b1,
           W2, b2, gamma, beta, Wr1, br1, Wr2, br2):
    n, d = x.shape
    num_layers = We.shape[0]
    c_out = Wr2.shape[1]
    node_blk = 1000
    edge_blk = 4000

    n_pad = 128 * ((n + 127) // 128)
    zeros_nd = jnp.zeros((n_pad, d), jnp.float32)
    batch3 = batch.reshape(n // node_blk, 1, node_blk)
    Wr2p = jnp.zeros((d, d), jnp.float32).at[:, :c_out].set(Wr2)
    br2p = jnp.zeros((1, d), jnp.float32).at[0, :c_out].set(br2)

    h = _mm_bias(x, W_in, b_in, node_blk)
    for l in range(num_layers):
        e = _mm_bias(edge_attr, We[l], be[l], edge_blk)
        parts = _sc_aggregate(h, e, edge_index, zeros_nd)
        z2, stats = _node_mlp_stats(h, parts, W1[l], b1[l], W2[l], b2[l],
                                    node_blk)
        h = _bn_relu(z2, stats, gamma[l], beta[l], node_blk)

    pooled = _pool(h, batch3, node_blk, _G)
    out = _readout(pooled, Wr1, br1, Wr2p, br2p)
    return out[:, :c_out]


# speedup vs baseline: 3.9167x; 1.2402x over previous
import functools

import jax
import jax.numpy as jnp
from jax import lax
from jax.experimental import pallas as pl
from jax.experimental.pallas import tpu as pltpu
from jax.experimental.pallas import tpu_sc as plsc

_NUM_SC = 2
_NUM_SUBCORES = 16
_LANES = 16
_G = 64


def _mm_bias(x, W, b, block_rows):
    n, k = x.shape
    m = W.shape[1]
    assert n % block_rows == 0

    def body(x_ref, w_ref, b_ref, o_ref):
        o_ref[...] = (
            jnp.dot(x_ref[...], w_ref[...], preferred_element_type=jnp.float32)
            + b_ref[...]
        )

    return pl.pallas_call(
        body,
        grid=(n // block_rows,),
        in_specs=[
            pl.BlockSpec((block_rows, k), lambda i: (i, 0)),
            pl.BlockSpec((k, m), lambda i: (0, 0)),
            pl.BlockSpec((1, m), lambda i: (0, 0)),
        ],
        out_specs=pl.BlockSpec((block_rows, m), lambda i: (i, 0)),
        out_shape=jax.ShapeDtypeStruct((n, m), jnp.float32),
    )(x, W, b.reshape(1, m))


def _node_mlp_stats(h, parts, W1, b1, W2, b2, block_rows):
    n, d = h.shape

    def body(h_ref, a0_ref, a1_ref, w1_ref, b1_ref, w2_ref, b2_ref,
             z2_ref, st_ref):
        z = h_ref[...] + a0_ref[0] + a1_ref[0]
        t = jnp.maximum(
            jnp.dot(z, w1_ref[...], preferred_element_type=jnp.float32)
            + b1_ref[...], 0.0)
        z2 = (jnp.dot(t, w2_ref[...], preferred_element_type=jnp.float32)
              + b2_ref[...])
        z2_ref[...] = z2
        s = jnp.sum(z2, axis=0, keepdims=True)
        ss = jnp.sum(z2 * z2, axis=0, keepdims=True)
        st = jnp.concatenate([s, ss, jnp.zeros((6, d), jnp.float32)], axis=0)

        @pl.when(pl.program_id(0) == 0)
        def _():
            st_ref[...] = st

        @pl.when(pl.program_id(0) != 0)
        def _():
            st_ref[...] += st

    return pl.pallas_call(
        body,
        grid=(n // block_rows,),
        in_specs=[
            pl.BlockSpec((block_rows, d), lambda i: (i, 0)),
            pl.BlockSpec((1, block_rows, d), lambda i: (0, i, 0)),
            pl.BlockSpec((1, block_rows, d), lambda i: (1, i, 0)),
            pl.BlockSpec((d, d), lambda i: (0, 0)),
            pl.BlockSpec((1, d), lambda i: (0, 0)),
            pl.BlockSpec((d, d), lambda i: (0, 0)),
            pl.BlockSpec((1, d), lambda i: (0, 0)),
        ],
        out_specs=[
            pl.BlockSpec((block_rows, d), lambda i: (i, 0)),
            pl.BlockSpec((8, d), lambda i: (0, 0)),
        ],
        out_shape=[
            jax.ShapeDtypeStruct((n, d), jnp.float32),
            jax.ShapeDtypeStruct((8, d), jnp.float32),
        ],
    )(h, parts, parts, W1, b1.reshape(1, d), W2, b2.reshape(1, d))


def _bn_relu(z2, stats, gamma_l, beta_l, block_rows):
    n, d = z2.shape
    inv_n = 1.0 / n

    def body(z_ref, st_ref, g_ref, b_ref, o_ref):
        mu = st_ref[0, :] * inv_n
        var = st_ref[1, :] * inv_n - mu * mu
        scale = g_ref[...] * lax.rsqrt(var + 1e-5)[None, :]
        o_ref[...] = jnp.maximum(
            (z_ref[...] - mu[None, :]) * scale + b_ref[...], 0.0)

    return pl.pallas_call(
        body,
        grid=(n // block_rows,),
        in_specs=[
            pl.BlockSpec((block_rows, d), lambda i: (i, 0)),
            pl.BlockSpec((8, d), lambda i: (0, 0)),
            pl.BlockSpec((1, d), lambda i: (0, 0)),
            pl.BlockSpec((1, d), lambda i: (0, 0)),
        ],
        out_specs=pl.BlockSpec((block_rows, d), lambda i: (i, 0)),
        out_shape=jax.ShapeDtypeStruct((n, d), jnp.float32),
    )(z2, stats, gamma_l.reshape(1, d), beta_l.reshape(1, d))


def _pool(h, batch3, block_rows, num_graphs):
    n, d = h.shape
    nblk = n // block_rows

    def body(h_ref, b_ref, o_ref):
        bi = b_ref[0, 0, :]
        oh = (bi[:, None]
              == lax.broadcasted_iota(jnp.int32, (block_rows, num_graphs), 1)
              ).astype(jnp.float32)
        contrib = lax.dot_general(
            oh, h_ref[...], (((0,), (0,)), ((), ())),
            preferred_element_type=jnp.float32)

        @pl.when(pl.program_id(0) == 0)
        def _():
            o_ref[...] = contrib

        @pl.when(pl.program_id(0) != 0)
        def _():
            o_ref[...] += contrib

    return pl.pallas_call(
        body,
        grid=(nblk,),
        in_specs=[
            pl.BlockSpec((block_rows, d), lambda i: (i, 0)),
            pl.BlockSpec((1, 1, block_rows), lambda i: (i, 0, 0)),
        ],
        out_specs=pl.BlockSpec((num_graphs, d), lambda i: (0, 0)),
        out_shape=jax.ShapeDtypeStruct((num_graphs, d), jnp.float32),
    )(h, batch3)


def _readout(pooled, Wr1, br1, Wr2p, br2p):
    g, d = pooled.shape

    def body(p_ref, w1_ref, b1_ref, w2_ref, b2_ref, o_ref):
        t = jnp.maximum(
            jnp.dot(p_ref[...], w1_ref[...], preferred_element_type=jnp.float32)
            + b1_ref[...], 0.0)
        o_ref[...] = (
            jnp.dot(t, w2_ref[...], preferred_element_type=jnp.float32)
            + b2_ref[...])

    return pl.pallas_call(
        body,
        grid=(1,),
        in_specs=[
            pl.BlockSpec((g, d), lambda i: (0, 0)),
            pl.BlockSpec((d, d), lambda i: (0, 0)),
            pl.BlockSpec((1, d), lambda i: (0, 0)),
            pl.BlockSpec((d, d), lambda i: (0, 0)),
            pl.BlockSpec((1, d), lambda i: (0, 0)),
        ],
        out_specs=pl.BlockSpec((g, d), lambda i: (0, 0)),
        out_shape=jax.ShapeDtypeStruct((g, d), jnp.float32),
    )(pooled, Wr1, br1.reshape(1, d), Wr2p, br2p)


def _sc_aggregate(h, e, src, dst, zeros_nd, K):
    n, d = h.shape
    n_pad = zeros_nd.shape[0]
    num_edges = src.shape[0]
    nw = _NUM_SC * _NUM_SUBCORES
    cpt = num_edges // (K * nw)
    pairs = cpt // 2
    rows_per_tile = n_pad // _NUM_SUBCORES

    mesh = plsc.VectorSubcoreMesh(
        core_axis_name="c", subcore_axis_name="s",
        num_cores=_NUM_SC, num_subcores=_NUM_SUBCORES)

    @functools.partial(
        pl.kernel,
        out_type=jax.ShapeDtypeStruct((_NUM_SC, n_pad, d), jnp.float32),
        mesh=mesh,
        scratch_types=[
            pltpu.VMEM_SHARED((n_pad, d), jnp.float32),
            pltpu.VMEM((K, d), jnp.float32),
            pltpu.VMEM((K, d), jnp.float32),
            pltpu.VMEM((K, d), jnp.float32),
            pltpu.VMEM((K, d), jnp.float32),
            pltpu.VMEM((K, d), jnp.float32),
            pltpu.VMEM((K, d), jnp.float32),
            pltpu.VMEM((K,), jnp.int32),
            pltpu.VMEM((K,), jnp.int32),
            pltpu.VMEM((K,), jnp.int32),
            pltpu.VMEM((K,), jnp.int32),
            pltpu.SemaphoreType.DMA,
            pltpu.SemaphoreType.DMA,
            pltpu.SemaphoreType.DMA,
            pltpu.SemaphoreType.DMA,
            pltpu.SemaphoreType.DMA,
            pltpu.SemaphoreType.DMA,
            pltpu.SemaphoreType.DMA,
            pltpu.SemaphoreType.DMA,
            pltpu.SemaphoreType.DMA,
            pltpu.SemaphoreType.DMA,
        ],
    )
    def agg(h_hbm, e_hbm, src_hbm, dst_hbm, z_hbm, out_hbm, acc,
            hrA, hrB, ebA, ebB, msA, msB, sxA, sxB, dxA, dxB,
            gsA, gsB, esA, esB, ssA, ssB, sxsA, sxsB, dxsA, dxsB):
        core = lax.axis_index("c")
        sub = lax.axis_index("s")
        wid = core * _NUM_SUBCORES + sub
        r0 = sub * rows_per_tile
        c0 = wid * cpt

        pltpu.sync_copy(z_hbm.at[pl.ds(r0, rows_per_tile)],
                        acc.at[pl.ds(r0, rows_per_tile)])
        plsc.subcore_barrier()

        def fetch_src(c_local, sx, sxs):
            pltpu.async_copy(src_hbm.at[pl.ds((c0 + c_local) * K, K)], sx, sxs)

        def fetch_dst(c_local, dx, dxs):
            pltpu.async_copy(dst_hbm.at[pl.ds((c0 + c_local) * K, K)], dx, dxs)

        def wait_idx(hbm, xb, xs):
            pltpu.make_async_copy(hbm.at[pl.ds(0, K)], xb, xs).wait()

        def issue_in(c_local, sx, hr, eb, gs, es):
            pltpu.async_copy(h_hbm.at[sx], hr, gs)
            pltpu.async_copy(e_hbm.at[pl.ds((c0 + c_local) * K, K)], eb, es)

        def wait_in(sx, hr, eb, gs, es):
            pltpu.make_async_copy(h_hbm.at[sx], hr, gs).wait()
            pltpu.make_async_copy(e_hbm.at[pl.ds(0, K)], eb, es).wait()

        def compute(hr, eb, ms):
            def row_body(r, carry):
                for q in range(d // _LANES):
                    s = pl.ds(q * _LANES, _LANES)
                    ms[r, s] = jnp.maximum(hr[r, s] + eb[r, s], 0.0)
                return carry

            lax.fori_loop(0, K, row_body, 0)

        def scatter(dx, ms, ss):
            pltpu.async_copy(ms, acc.at[dx], ss, add=True)

        def wait_scat(dx, ms, ss):
            pltpu.make_async_copy(ms, acc.at[dx], ss).wait()

        fetch_src(0, sxA, sxsA)
        fetch_src(1, sxB, sxsB)
        wait_idx(src_hbm, sxA, sxsA)
        issue_in(0, sxA, hrA, ebA, gsA, esA)
        wait_idx(src_hbm, sxB, sxsB)
        issue_in(1, sxB, hrB, ebB, gsB, esB)

        def slot_step(p, c, sx, dx, hr, eb, ms, gs, es, ss, sxs, dxs):
            wait_in(sx, hr, eb, gs, es)

            @pl.when(p < pairs - 1)
            def _():
                fetch_src(c + 2, sx, sxs)

            @pl.when(p > 0)
            def _():
                wait_scat(dx, ms, ss)

            fetch_dst(c, dx, dxs)
            compute(hr, eb, ms)

            @pl.when(p < pairs - 1)
            def _():
                wait_idx(src_hbm, sx, sxs)
                issue_in(c + 2, sx, hr, eb, gs, es)

            wait_idx(dst_hbm, dx, dxs)
            scatter(dx, ms, ss)

        def pair_body(p, carry):
            slot_step(p, 2 * p, sxA, dxA, hrA, ebA, msA, gsA, esA, ssA,
                      sxsA, dxsA)
            slot_step(p, 2 * p + 1, sxB, dxB, hrB, ebB, msB, gsB, esB, ssB,
                      sxsB, dxsB)
            return carry

        lax.fori_loop(0, pairs, pair_body, 0)
        wait_scat(dxA, msA, ssA)
        wait_scat(dxB, msB, ssB)
        plsc.subcore_barrier()

        pltpu.sync_copy(acc.at[pl.ds(r0, rows_per_tile)],
                        out_hbm.at[core, pl.ds(r0, rows_per_tile)])

    return agg(h, e, src, dst, zeros_nd)


def kernel(x, edge_index, edge_attr, batch, W_in, b_in, We, be, W1, b1,
           W2, b2, gamma, beta, Wr1, br1, Wr2, br2):
    n, d = x.shape
    num_edges = edge_index.shape[1]
    de = edge_attr.shape[1]
    num_layers = We.shape[0]
    c_out = Wr2.shape[1]
    node_blk = 1000
    K = 56
    nw = _NUM_SC * _NUM_SUBCORES
    edge_blk = 2 * K * nw

    n_pad = 128 * ((n + 127) // 128)
    cpt = -(-num_edges // (K * nw))
    cpt += cpt % 2
    e_pad = cpt * K * nw - num_edges

    zeros_nd = jnp.zeros((n_pad, d), jnp.float32)
    pad_idx = jnp.stack([jnp.zeros((e_pad,), jnp.int32),
                         jnp.full((e_pad,), n_pad - 1, jnp.int32)])
    ei_pad = jnp.concatenate([edge_index, pad_idx], axis=1)
    src_pad = ei_pad[0]
    dst_pad = ei_pad[1]
    ea_pad = jnp.concatenate(
        [edge_attr, jnp.zeros((e_pad, de), jnp.float32)], axis=0)
    batch3 = batch.reshape(n // node_blk, 1, node_blk)
    Wr2p = jnp.zeros((d, d), jnp.float32).at[:, :c_out].set(Wr2)
    br2p = jnp.zeros((1, d), jnp.float32).at[0, :c_out].set(br2)

    h = _mm_bias(x, W_in, b_in, node_blk)
    for l in range(num_layers):
        e = _mm_bias(ea_pad, We[l], be[l], edge_blk)
        parts = _sc_aggregate(h, e, src_pad, dst_pad, zeros_nd, K)
        z2, stats = _node_mlp_stats(h, parts, W1[l], b1[l], W2[l], b2[l],
                                    node_blk)
        h = _bn_relu(z2, stats, gamma[l], beta[l], node_blk)

    pooled = _pool(h, batch3, node_blk, _G)
    out = _readout(pooled, Wr1, br1, Wr2p, br2p)
    return out[:, :c_out]
